# baseline (speedup 1.0000x reference)
import functools

import jax
import jax.numpy as jnp
from jax import lax
from jax.experimental import pallas as pl
from jax.experimental.pallas import tpu as pltpu
from jax.experimental.pallas import tpu_sc as plsc

N = 10000
D = 128
HD = D // 2
E = 320000
NC = 2
NS = 16
NW = NC * NS
CHUNK = 128
EPAD = NW * 80 * CHUNK
NCH_D = EPAD // (NW * CHUNK)
NCH_S = EPAD // (NS * CHUNK)
RPS = 624
TAIL = N - NS * RPS

_mesh = plsc.VectorSubcoreMesh(
    core_axis_name="c", subcore_axis_name="s", num_cores=NC, num_subcores=NS
)


HPAD = 10112


def _deg_body(dstr, degp, dst_v, hist):
    c = lax.axis_index("c")
    s = lax.axis_index("s")
    wid = c * NS + s
    z = jnp.zeros((16,), jnp.float32)

    def zb(i, _):
        hist[pl.ds(i * 16, 16)] = z
        return 0

    lax.fori_loop(0, HPAD // 16, zb, 0)
    pltpu.sync_copy(dstr.at[wid], dst_v)
    ones = jnp.full((16,), 1.0, jnp.float32)

    def body(r, _):
        for j in range(CHUNK // 16):
            iv = dst_v[r, pl.ds(j * 16, 16)]
            plsc.addupdate_scatter(hist, [iv], ones)
        return 0

    lax.fori_loop(0, NCH_D, body, 0)
    pltpu.sync_copy(hist, degp.at[wid])


SSTEP = 104
NBUF = 4


def _scat_body(y2, srcr, dstr, accs, src_v, dst_v, rows_v, acc_sp, gsem, ssem):
    c = lax.axis_index("c")
    s = lax.axis_index("s")
    base = s * RPS
    stg = rows_v.at[0, pl.ds(0, SSTEP)]
    stg_t = rows_v.at[0, pl.ds(0, TAIL)]

    for k in range(RPS // SSTEP):
        pltpu.sync_copy(y2.at[c, pl.ds(base + k * SSTEP, SSTEP)], stg)
        pltpu.sync_copy(stg, acc_sp.at[pl.ds(base + k * SSTEP, SSTEP)])

    @pl.when(s == NS - 1)
    def _():
        pltpu.sync_copy(y2.at[c, pl.ds(N - TAIL, TAIL)], stg_t)
        pltpu.sync_copy(stg_t, acc_sp.at[pl.ds(N - TAIL, TAIL)])

    plsc.subcore_barrier()

    pltpu.sync_copy(srcr.at[s], src_v)
    pltpu.sync_copy(dstr.at[s], dst_v)

    def _g_start(i):
        buf = rows_v.at[jnp.bitwise_and(i, NBUF - 1)]

        @pl.when(c == 0)
        def _():
            pltpu.make_async_copy(y2.at[0].at[src_v.at[i]], buf, gsem).start()

        @pl.when(c != 0)
        def _():
            pltpu.make_async_copy(y2.at[1].at[src_v.at[i]], buf, gsem).start()

    def _g_wait(i):
        buf = rows_v.at[jnp.bitwise_and(i, NBUF - 1)]

        @pl.when(c == 0)
        def _():
            pltpu.make_async_copy(y2.at[0].at[src_v.at[i]], buf, gsem).wait()

        @pl.when(c != 0)
        def _():
            pltpu.make_async_copy(y2.at[1].at[src_v.at[i]], buf, gsem).wait()

    def _s_start(i):
        pltpu.make_async_copy(
            rows_v.at[jnp.bitwise_and(i, NBUF - 1)], acc_sp.at[dst_v.at[i]], ssem
        ).start(add=True)

    def _s_wait(i):
        pltpu.make_async_copy(
            rows_v.at[jnp.bitwise_and(i, NBUF - 1)], acc_sp.at[dst_v.at[i]], ssem
        ).wait()

    def prol(i, _):
        _g_start(i)
        return 0

    lax.fori_loop(0, NBUF - 1, prol, 0)

    def body(i, _):
        _g_wait(i)
        _s_start(i)

        @pl.when(i >= 1)
        def _():
            _s_wait(i - 1)

        @pl.when(i + NBUF - 1 < NCH_S)
        def _():
            _g_start(i + NBUF - 1)

        return 0

    lax.fori_loop(0, NCH_S, body, 0)
    _s_wait(NCH_S - 1)
    plsc.subcore_barrier()

    for k in range(RPS // SSTEP):
        pltpu.sync_copy(acc_sp.at[pl.ds(base + k * SSTEP, SSTEP)], stg)
        pltpu.sync_copy(stg, accs.at[c, pl.ds(base + k * SSTEP, SSTEP)])

    @pl.when(s == NS - 1)
    def _():
        pltpu.sync_copy(acc_sp.at[pl.ds(N - TAIL, TAIL)], stg_t)
        pltpu.sync_copy(stg_t, accs.at[c, pl.ds(N - TAIL, TAIL)])


def _make_deg_kernel(interpret=False):
    return pl.kernel(
        _deg_body,
        mesh=_mesh,
        out_type=jax.ShapeDtypeStruct((NW, HPAD), jnp.float32),
        scratch_types=[
            pltpu.VMEM((NCH_D, CHUNK), jnp.int32),
            pltpu.VMEM((HPAD,), jnp.float32),
        ],
        compiler_params=pltpu.CompilerParams(needs_layout_passes=False),
        interpret=interpret,
    )


def _make_scat_kernel(interpret=False):
    return pl.kernel(
        _scat_body,
        mesh=_mesh,
        out_type=jax.ShapeDtypeStruct((NC, N, HD), jnp.float32),
        scratch_types=[
            pltpu.VMEM((NCH_S, CHUNK), jnp.int32),
            pltpu.VMEM((NCH_S, CHUNK), jnp.int32),
            pltpu.VMEM((NBUF, CHUNK, HD), jnp.float32),
            pltpu.VMEM_SHARED((N + 8, HD), jnp.float32),
            pltpu.SemaphoreType.DMA,
            pltpu.SemaphoreType.DMA,
        ],
        compiler_params=pltpu.CompilerParams(use_tc_tiling_on_sc=False),
        interpret=interpret,
    )


_deg_kernel = _make_deg_kernel()
_scat_kernel = _make_scat_kernel()


BS = 1024
GRID = (N + BS - 1) // BS


def _dinv_of(degp_blk):
    ones = jnp.ones((NW, 1), jnp.float32)
    colsum = lax.dot_general(
        degp_blk, ones, (((0,), (0,)), ((), ())),
        preferred_element_type=jnp.float32,
    )
    return lax.rsqrt(colsum + 1.0)


def _split_store(y_ref, yb):
    y_ref[0, :, :] = yb[:, :HD]
    y_ref[1, :, :] = yb[:, HD:]


def _first_body(x_ref, w_ref, degs_ref, y_ref):
    dinv = _dinv_of(degs_ref[...])
    yb = jnp.dot(x_ref[...], w_ref[...], preferred_element_type=jnp.float32) * dinv
    _split_store(y_ref, yb)


def _first(x, W1, degp):
    return pl.pallas_call(
        _first_body,
        grid=(GRID,),
        in_specs=[
            pl.BlockSpec((BS, D), lambda i: (i, 0)),
            pl.BlockSpec((D, D), lambda i: (0, 0)),
            pl.BlockSpec((NW, BS), lambda i: (0, i)),
        ],
        out_specs=pl.BlockSpec((NC, BS, HD), lambda i: (0, i, 0)),
        out_shape=jax.ShapeDtypeStruct((NC, N, HD), jnp.float32),
    )(x, W1, degp)


def _mid_body(accs_ref, degs_ref, b_ref, w_ref, y_ref):
    dinv = _dinv_of(degs_ref[...])
    acc = jnp.concatenate([accs_ref[0], accs_ref[1]], axis=1)
    h = jnp.maximum(acc * dinv + b_ref[...], 0.0)
    yb = jnp.dot(h, w_ref[...], preferred_element_type=jnp.float32) * dinv
    _split_store(y_ref, yb)


def _mid(accs, degp, b, Wn):
    return pl.pallas_call(
        _mid_body,
        grid=(GRID,),
        in_specs=[
            pl.BlockSpec((NC, BS, HD), lambda i: (0, i, 0)),
            pl.BlockSpec((NW, BS), lambda i: (0, i)),
            pl.BlockSpec((1, D), lambda i: (0, 0)),
            pl.BlockSpec((D, D), lambda i: (0, 0)),
        ],
        out_specs=pl.BlockSpec((NC, BS, HD), lambda i: (0, i, 0)),
        out_shape=jax.ShapeDtypeStruct((NC, N, HD), jnp.float32),
    )(accs, degp, b, Wn)


def _final_body(accs_ref, degs_ref, b_ref, y_ref):
    dinv = _dinv_of(degs_ref[...])
    acc = jnp.concatenate([accs_ref[0], accs_ref[1]], axis=1)
    y_ref[...] = acc * dinv + b_ref[...]


def _final(accs, degp, b):
    return pl.pallas_call(
        _final_body,
        grid=(GRID,),
        in_specs=[
            pl.BlockSpec((NC, BS, HD), lambda i: (0, i, 0)),
            pl.BlockSpec((NW, BS), lambda i: (0, i)),
            pl.BlockSpec((1, D), lambda i: (0, 0)),
        ],
        out_specs=pl.BlockSpec((BS, D), lambda i: (i, 0)),
        out_shape=jax.ShapeDtypeStruct((N, D), jnp.float32),
    )(accs, degp, b)


def kernel(x, edge_index, W1, b1, W2, b2, W3, b3):
    pad = EPAD - E
    src = jnp.concatenate(
        [edge_index[0].astype(jnp.int32), jnp.zeros((pad,), jnp.int32)]
    )
    dst = jnp.concatenate(
        [edge_index[1].astype(jnp.int32), jnp.full((pad,), N, jnp.int32)]
    )
    src_d = src.reshape(NW, NCH_D, CHUNK)
    dst_d = dst.reshape(NW, NCH_D, CHUNK)
    src_s = src.reshape(NS, NCH_S, CHUNK)
    dst_s = dst.reshape(NS, NCH_S, CHUNK)

    degp = _deg_kernel(dst_d)
    y0 = _first(x, W1, degp)

    Wstack = jnp.stack([W2, W3, W3])
    bstack = jnp.stack([b1.reshape(1, D), b2.reshape(1, D), b3.reshape(1, D)])

    def step(carry, wb):
        y, _ = carry
        Wn, bn = wb
        accs = _scat_kernel(y, src_s, dst_s)
        yn = _mid(accs, degp, bn, Wn)
        return (yn, accs), None

    accs0 = jnp.zeros((NC, N, HD), jnp.float32)
    (_, accs), _ = lax.scan(step, (y0, accs0), (Wstack, bstack))
    return _final(accs, degp, b3.reshape(1, D))


# speedup vs baseline: 1.0189x; 1.0189x over previous
import functools

import jax
import jax.numpy as jnp
from jax import lax
from jax.experimental import pallas as pl
from jax.experimental.pallas import tpu as pltpu
from jax.experimental.pallas import tpu_sc as plsc

N = 10000
D = 128
HD = D // 2
E = 320000
NC = 2
NS = 16
NW = NC * NS
CHUNK = 128
EPAD = NW * 80 * CHUNK
NCH_D = EPAD // (NW * CHUNK)
NCH_S = EPAD // (NS * CHUNK)
RPS = 624
TAIL = N - NS * RPS

_mesh = plsc.VectorSubcoreMesh(
    core_axis_name="c", subcore_axis_name="s", num_cores=NC, num_subcores=NS
)


HPAD = 10112


def _deg_body(dstr, degp, dst_v, hist):
    c = lax.axis_index("c")
    s = lax.axis_index("s")
    wid = c * NS + s
    z = jnp.zeros((16,), jnp.float32)

    def zb(i, _):
        hist[pl.ds(i * 16, 16)] = z
        return 0

    lax.fori_loop(0, HPAD // 16, zb, 0)
    pltpu.sync_copy(dstr.at[wid], dst_v)
    ones = jnp.full((16,), 1.0, jnp.float32)

    def body(r, _):
        for j in range(CHUNK // 16):
            iv = dst_v[r, pl.ds(j * 16, 16)]
            plsc.addupdate_scatter(hist, [iv], ones)
        return 0

    lax.fori_loop(0, NCH_D, body, 0)
    pltpu.sync_copy(hist, degp.at[wid])


SSTEP = 104
NBUF = 4


def _scat_body(y2, srcr, dstr, accs, src_v, dst_v, rows_v, acc_sp, gsem, ssem):
    c = lax.axis_index("c")
    s = lax.axis_index("s")
    base = s * RPS
    stg = rows_v.at[0, pl.ds(0, SSTEP)]
    stg_t = rows_v.at[0, pl.ds(0, TAIL)]

    for k in range(RPS // SSTEP):
        pltpu.sync_copy(y2.at[c, pl.ds(base + k * SSTEP, SSTEP)], stg)
        pltpu.sync_copy(stg, acc_sp.at[pl.ds(base + k * SSTEP, SSTEP)])

    @pl.when(s == NS - 1)
    def _():
        pltpu.sync_copy(y2.at[c, pl.ds(N - TAIL, TAIL)], stg_t)
        pltpu.sync_copy(stg_t, acc_sp.at[pl.ds(N - TAIL, TAIL)])

    plsc.subcore_barrier()

    pltpu.sync_copy(srcr.at[s], src_v)
    pltpu.sync_copy(dstr.at[s], dst_v)

    def _g_start(i):
        buf = rows_v.at[jnp.bitwise_and(i, NBUF - 1)]

        @pl.when(c == 0)
        def _():
            pltpu.make_async_copy(y2.at[0].at[src_v.at[i]], buf, gsem).start()

        @pl.when(c != 0)
        def _():
            pltpu.make_async_copy(y2.at[1].at[src_v.at[i]], buf, gsem).start()

    def _g_wait(i):
        buf = rows_v.at[jnp.bitwise_and(i, NBUF - 1)]

        @pl.when(c == 0)
        def _():
            pltpu.make_async_copy(y2.at[0].at[src_v.at[i]], buf, gsem).wait()

        @pl.when(c != 0)
        def _():
            pltpu.make_async_copy(y2.at[1].at[src_v.at[i]], buf, gsem).wait()

    def _s_start(i):
        pltpu.make_async_copy(
            rows_v.at[jnp.bitwise_and(i, NBUF - 1)], acc_sp.at[dst_v.at[i]], ssem
        ).start(add=True)

    def _s_wait(i):
        pltpu.make_async_copy(
            rows_v.at[jnp.bitwise_and(i, NBUF - 1)], acc_sp.at[dst_v.at[i]], ssem
        ).wait()

    def prol(i, _):
        _g_start(i)
        return 0

    lax.fori_loop(0, NBUF - 1, prol, 0)

    def body(i, _):
        _g_wait(i)
        _s_start(i)

        @pl.when(i >= 1)
        def _():
            _s_wait(i - 1)

        @pl.when(i + NBUF - 1 < NCH_S)
        def _():
            _g_start(i + NBUF - 1)

        return 0

    lax.fori_loop(0, NCH_S, body, 0)
    _s_wait(NCH_S - 1)
    plsc.subcore_barrier()

    for k in range(RPS // SSTEP):
        pltpu.sync_copy(acc_sp.at[pl.ds(base + k * SSTEP, SSTEP)], stg)
        pltpu.sync_copy(stg, accs.at[c, pl.ds(base + k * SSTEP, SSTEP)])

    @pl.when(s == NS - 1)
    def _():
        pltpu.sync_copy(acc_sp.at[pl.ds(N - TAIL, TAIL)], stg_t)
        pltpu.sync_copy(stg_t, accs.at[c, pl.ds(N - TAIL, TAIL)])


def _make_deg_kernel(interpret=False):
    return pl.kernel(
        _deg_body,
        mesh=_mesh,
        out_type=jax.ShapeDtypeStruct((NW, HPAD), jnp.float32),
        scratch_types=[
            pltpu.VMEM((NCH_D, CHUNK), jnp.int32),
            pltpu.VMEM((HPAD,), jnp.float32),
        ],
        compiler_params=pltpu.CompilerParams(needs_layout_passes=False),
        interpret=interpret,
    )


def _make_scat_kernel(interpret=False):
    return pl.kernel(
        _scat_body,
        mesh=_mesh,
        out_type=jax.ShapeDtypeStruct((NC, N, HD), jnp.float32),
        scratch_types=[
            pltpu.VMEM((NCH_S, CHUNK), jnp.int32),
            pltpu.VMEM((NCH_S, CHUNK), jnp.int32),
            pltpu.VMEM((NBUF, CHUNK, HD), jnp.float32),
            pltpu.VMEM_SHARED((N + 8, HD), jnp.float32),
            pltpu.SemaphoreType.DMA,
            pltpu.SemaphoreType.DMA,
        ],
        compiler_params=pltpu.CompilerParams(use_tc_tiling_on_sc=False),
        interpret=interpret,
    )


_deg_kernel = _make_deg_kernel()
_scat_kernel = _make_scat_kernel()


BS = 1024
GRID = (N + BS - 1) // BS


def _dinv_of(degp_blk):
    ones = jnp.ones((NW, 1), jnp.float32)
    colsum = lax.dot_general(
        degp_blk, ones, (((0,), (0,)), ((), ())),
        preferred_element_type=jnp.float32,
    )
    return lax.rsqrt(colsum + 1.0)


def _split_store(y_ref, yb):
    y_ref[0, :, :] = yb[:, :HD]
    y_ref[1, :, :] = yb[:, HD:]


def _first_body(x_ref, w_ref, degs_ref, y_ref):
    dinv = _dinv_of(degs_ref[...])
    yb = jnp.dot(x_ref[...], w_ref[...], preferred_element_type=jnp.float32) * dinv
    _split_store(y_ref, yb)


def _first(x, W1, degp):
    return pl.pallas_call(
        _first_body,
        grid=(GRID,),
        in_specs=[
            pl.BlockSpec((BS, D), lambda i: (i, 0)),
            pl.BlockSpec((D, D), lambda i: (0, 0)),
            pl.BlockSpec((NW, BS), lambda i: (0, i)),
        ],
        out_specs=pl.BlockSpec((NC, BS, HD), lambda i: (0, i, 0)),
        out_shape=jax.ShapeDtypeStruct((NC, N, HD), jnp.float32),
    )(x, W1, degp)


def _mid_body(accs_ref, degs_ref, b_ref, w_ref, y_ref):
    dinv = _dinv_of(degs_ref[...])
    acc = jnp.concatenate([accs_ref[0], accs_ref[1]], axis=1)
    h = jnp.maximum(acc * dinv + b_ref[...], 0.0)
    yb = jnp.dot(h, w_ref[...], preferred_element_type=jnp.float32) * dinv
    _split_store(y_ref, yb)


def _mid(accs, degp, b, Wn):
    return pl.pallas_call(
        _mid_body,
        grid=(GRID,),
        in_specs=[
            pl.BlockSpec((NC, BS, HD), lambda i: (0, i, 0)),
            pl.BlockSpec((NW, BS), lambda i: (0, i)),
            pl.BlockSpec((1, D), lambda i: (0, 0)),
            pl.BlockSpec((D, D), lambda i: (0, 0)),
        ],
        out_specs=pl.BlockSpec((NC, BS, HD), lambda i: (0, i, 0)),
        out_shape=jax.ShapeDtypeStruct((NC, N, HD), jnp.float32),
    )(accs, degp, b, Wn)


def _final_body(accs_ref, degs_ref, b_ref, y_ref):
    dinv = _dinv_of(degs_ref[...])
    acc = jnp.concatenate([accs_ref[0], accs_ref[1]], axis=1)
    y_ref[...] = acc * dinv + b_ref[...]


def _final(accs, degp, b):
    return pl.pallas_call(
        _final_body,
        grid=(GRID,),
        in_specs=[
            pl.BlockSpec((NC, BS, HD), lambda i: (0, i, 0)),
            pl.BlockSpec((NW, BS), lambda i: (0, i)),
            pl.BlockSpec((1, D), lambda i: (0, 0)),
        ],
        out_specs=pl.BlockSpec((BS, D), lambda i: (i, 0)),
        out_shape=jax.ShapeDtypeStruct((N, D), jnp.float32),
    )(accs, degp, b)


def kernel(x, edge_index, W1, b1, W2, b2, W3, b3):
    pad = EPAD - E
    src = jnp.concatenate(
        [edge_index[0].astype(jnp.int32), jnp.zeros((pad,), jnp.int32)]
    )
    dst = jnp.concatenate(
        [edge_index[1].astype(jnp.int32), jnp.full((pad,), N, jnp.int32)]
    )
    src_d = src.reshape(NW, NCH_D, CHUNK)
    dst_d = dst.reshape(NW, NCH_D, CHUNK)
    src_s = src.reshape(NS, NCH_S, CHUNK)
    dst_s = dst.reshape(NS, NCH_S, CHUNK)

    degp = _deg_kernel(dst_d)
    y = _first(x, W1, degp)
    accs = _scat_kernel(y, src_s, dst_s)
    y = _mid(accs, degp, b1.reshape(1, D), W2)
    accs = _scat_kernel(y, src_s, dst_s)
    y = _mid(accs, degp, b2.reshape(1, D), W3)
    accs = _scat_kernel(y, src_s, dst_s)
    return _final(accs, degp, b3.reshape(1, D))
